# Initial kernel scaffold; baseline (speedup 1.0000x reference)
import jax
import jax.numpy as jnp
from jax.experimental import pallas as pl


def kernel(Z, Dij, Qa, idx_i, idx_j):
    raise NotImplementedError("write your pallas kernel here")


# speedup vs baseline: 114.6820x; 114.6820x over previous
import functools

import jax
import jax.numpy as jnp
from jax import lax
from jax.experimental import pallas as pl
from jax.experimental.pallas import tpu as pltpu, tpu_sc as plsc

N_NODES = 50000
N_EDGES = 1600000
NUM_CORES = 2
NUM_SUBCORES = 16
NW = NUM_CORES * NUM_SUBCORES
EDGES_PER_WORKER = N_EDGES // NW
CHUNK = 2000
NCHUNKS = EDGES_PER_WORKER // CHUNK
VECS = CHUNK // 16
CUT = 5.0
INV_CUT = 1.0 / CUT


def _rsqrt16(a):
    bi = plsc.bitcast(a, jnp.int32)
    yi = jnp.int32(0x5F3759DF) - (bi >> 1)
    y = plsc.bitcast(yi, jnp.float32)
    for _ in range(3):
        y = y * (1.5 - 0.5 * a * y * y)
    return y


def _sc_body(dij_hbm, qa_hbm, idxi_hbm, idxj_hbm, out_hbm,
             qa_v, acc_v, dij_v, idxi_v, idxj_v):
    wid = lax.axis_index("c") * NUM_SUBCORES + lax.axis_index("s")
    base = wid * EDGES_PER_WORKER

    pltpu.sync_copy(qa_hbm, qa_v)

    zeros16 = jnp.zeros((16,), jnp.float32)

    @pl.loop(0, N_NODES // 16)
    def _zero(i):
        acc_v[pl.ds(i * 16, 16)] = zeros16

    @pl.loop(0, NCHUNKS)
    def _chunk(k):
        off = base + k * CHUNK
        pltpu.sync_copy(dij_hbm.at[pl.ds(off, CHUNK)], dij_v)
        pltpu.sync_copy(idxi_hbm.at[pl.ds(off, CHUNK)], idxi_v)
        pltpu.sync_copy(idxj_hbm.at[pl.ds(off, CHUNK)], idxj_v)

        @pl.loop(0, VECS)
        def _vec(j):
            sl = pl.ds(j * 16, 16)
            d = dij_v[sl]
            ii = idxi_v[sl]
            jj = idxj_v[sl]
            qi = plsc.load_gather(qa_v, [ii])
            qj = plsc.load_gather(qa_v, [jj])
            x = d * INV_CUT
            x3 = x * x * x
            poly = x3 * ((6.0 * x - 15.0) * x + 10.0)
            sw = jnp.where(d < CUT, poly, 1.0)
            e_ord = 1.0 / d
            e_shield = _rsqrt16(d * d + 1.0)
            e = (0.5 * qi) * qj * ((1.0 - sw) * e_shield + sw * e_ord)
            plsc.addupdate_scatter(acc_v, [ii], e)

    pltpu.sync_copy(acc_v, out_hbm.at[wid])


_sc_partials = pl.kernel(
    _sc_body,
    out_type=jax.ShapeDtypeStruct((NW, N_NODES), jnp.float32),
    mesh=plsc.VectorSubcoreMesh(
        core_axis_name="c", subcore_axis_name="s",
        num_cores=NUM_CORES, num_subcores=NUM_SUBCORES),
    scratch_types=[
        pltpu.VMEM((N_NODES,), jnp.float32),
        pltpu.VMEM((N_NODES,), jnp.float32),
        pltpu.VMEM((CHUNK,), jnp.float32),
        pltpu.VMEM((CHUNK,), jnp.int32),
        pltpu.VMEM((CHUNK,), jnp.int32),
    ],
    compiler_params=pltpu.CompilerParams(needs_layout_passes=False),
)


def _merge_body(x_ref, o_ref):
    o_ref[...] = jnp.sum(x_ref[...], axis=0)


_merge = pl.pallas_call(
    _merge_body,
    out_shape=jax.ShapeDtypeStruct((N_NODES,), jnp.float32),
)


@jax.jit
def kernel(Z, Dij, Qa, idx_i, idx_j):
    partials = _sc_partials(Dij, Qa, idx_i, idx_j)
    eele = _merge(partials)
    return (eele, Qa)


# speedup vs baseline: 257.8723x; 2.2486x over previous
import functools

import jax
import jax.numpy as jnp
from jax import lax
from jax.experimental import pallas as pl
from jax.experimental.pallas import tpu as pltpu, tpu_sc as plsc

N_NODES = 50000
N_EDGES = 1600000
NUM_CORES = 2
NUM_SUBCORES = 16
NW = NUM_CORES * NUM_SUBCORES
EDGES_PER_WORKER = N_EDGES // NW
CHUNK = 2000
NCHUNKS = EDGES_PER_WORKER // CHUNK
VECS = CHUNK // 16
CUT = 5.0
INV_CUT = 1.0 / CUT


def _rsqrt16(a):
    bi = plsc.bitcast(a, jnp.int32)
    yi = jnp.int32(0x5F3759DF) - (bi >> 1)
    y = plsc.bitcast(yi, jnp.float32)
    for _ in range(3):
        y = y * (1.5 - 0.5 * a * y * y)
    return y


def _sc_body(dij_hbm, qa_hbm, idxi_hbm, idxj_hbm, out_hbm,
             qa_v, acc_v, dij0, dij1, idxi0, idxi1, idxj0, idxj1,
             sem0, sem1, qsem):
    wid = lax.axis_index("c") * NUM_SUBCORES + lax.axis_index("s")
    base = wid * EDGES_PER_WORKER
    bufs = ((dij0, idxi0, idxj0, sem0), (dij1, idxi1, idxj1, sem1))

    def fire(k, slot):
        off = base + k * CHUNK
        d_v, i_v, j_v, s = bufs[slot]
        pltpu.async_copy(dij_hbm.at[pl.ds(off, CHUNK)], d_v, s)
        pltpu.async_copy(idxi_hbm.at[pl.ds(off, CHUNK)], i_v, s)
        pltpu.async_copy(idxj_hbm.at[pl.ds(off, CHUNK)], j_v, s)

    def drain(slot):
        d_v, i_v, j_v, s = bufs[slot]
        pltpu.make_async_copy(dij_hbm.at[pl.ds(0, CHUNK)], d_v, s).wait()
        pltpu.make_async_copy(idxi_hbm.at[pl.ds(0, CHUNK)], i_v, s).wait()
        pltpu.make_async_copy(idxj_hbm.at[pl.ds(0, CHUNK)], j_v, s).wait()

    def compute(slot):
        d_v, i_v, j_v, _ = bufs[slot]

        @plsc.parallel_loop(0, VECS, unroll=4)
        def _vec(j):
            sl = pl.ds(j * 16, 16)
            d = d_v[sl]
            ii = i_v[sl]
            jj = j_v[sl]
            qi = plsc.load_gather(qa_v, [ii])
            qj = plsc.load_gather(qa_v, [jj])
            x = d * INV_CUT
            x3 = x * x * x
            poly = x3 * ((6.0 * x - 15.0) * x + 10.0)
            sw = jnp.where(d < CUT, poly, 1.0)
            e_ord = 1.0 / d
            e_shield = _rsqrt16(d * d + 1.0)
            e = (0.5 * qi) * qj * ((1.0 - sw) * e_shield + sw * e_ord)
            plsc.addupdate_scatter(acc_v, [ii], e)

    pltpu.async_copy(qa_hbm, qa_v, qsem)
    fire(0, 0)
    zeros16 = jnp.zeros((16,), jnp.float32)

    @pl.loop(0, N_NODES // 16)
    def _zero(i):
        acc_v[pl.ds(i * 16, 16)] = zeros16

    pltpu.make_async_copy(qa_hbm, qa_v, qsem).wait()

    @pl.loop(0, (NCHUNKS - 1) // 2)
    def _pair(t):
        k = t * 2
        drain(0)
        fire(k + 1, 1)
        compute(0)
        drain(1)
        fire(k + 2, 0)
        compute(1)

    drain(0)
    compute(0)

    pltpu.sync_copy(acc_v, out_hbm.at[wid])


_sc_partials = pl.kernel(
    _sc_body,
    out_type=jax.ShapeDtypeStruct((NW, N_NODES), jnp.float32),
    mesh=plsc.VectorSubcoreMesh(
        core_axis_name="c", subcore_axis_name="s",
        num_cores=NUM_CORES, num_subcores=NUM_SUBCORES),
    scratch_types=[
        pltpu.VMEM((N_NODES,), jnp.float32),
        pltpu.VMEM((N_NODES,), jnp.float32),
        pltpu.VMEM((CHUNK,), jnp.float32),
        pltpu.VMEM((CHUNK,), jnp.float32),
        pltpu.VMEM((CHUNK,), jnp.int32),
        pltpu.VMEM((CHUNK,), jnp.int32),
        pltpu.VMEM((CHUNK,), jnp.int32),
        pltpu.VMEM((CHUNK,), jnp.int32),
        pltpu.SemaphoreType.DMA,
        pltpu.SemaphoreType.DMA,
        pltpu.SemaphoreType.DMA,
    ],
    compiler_params=pltpu.CompilerParams(needs_layout_passes=False),
)


def _merge_body(x_ref, o_ref):
    o_ref[...] = jnp.sum(x_ref[...], axis=0)


_merge = pl.pallas_call(
    _merge_body,
    out_shape=jax.ShapeDtypeStruct((N_NODES,), jnp.float32),
)


@jax.jit
def kernel(Z, Dij, Qa, idx_i, idx_j):
    partials = _sc_partials(Dij, Qa, idx_i, idx_j)
    eele = _merge(partials)
    return (eele, Qa)


# speedup vs baseline: 391.3349x; 1.5176x over previous
import functools

import jax
import jax.numpy as jnp
from jax import lax
from jax.experimental import pallas as pl
from jax.experimental.pallas import tpu as pltpu, tpu_sc as plsc

N_NODES = 50000
N_EDGES = 1600000
NUM_CORES = 2
NUM_SUBCORES = 16
NW = NUM_CORES * NUM_SUBCORES
EDGES_PER_WORKER = N_EDGES // NW
CHUNK = 2000
NCHUNKS = EDGES_PER_WORKER // CHUNK
VECS = CHUNK // 16
CUT = 5.0
INV_CUT = 1.0 / CUT


def _rsqrt16(a):
    bi = plsc.bitcast(a, jnp.int32)
    yi = jnp.int32(0x5F3759DF) - (bi >> 1)
    y = plsc.bitcast(yi, jnp.float32)
    for _ in range(2):
        y = y * (1.5 - 0.5 * a * y * y)
    return y


def _sc_body(dij_hbm, qa_hbm, idxi_hbm, idxj_hbm, out_hbm,
             qa_v, acc_v, dij0, dij1, idxi0, idxi1, idxj0, idxj1,
             sem0, sem1, qsem):
    wid = lax.axis_index("c") * NUM_SUBCORES + lax.axis_index("s")
    base = wid * EDGES_PER_WORKER
    bufs = ((dij0, idxi0, idxj0, sem0), (dij1, idxi1, idxj1, sem1))

    def fire(k, slot):
        off = base + k * CHUNK
        d_v, i_v, j_v, s = bufs[slot]
        pltpu.async_copy(dij_hbm.at[pl.ds(off, CHUNK)], d_v, s)
        pltpu.async_copy(idxi_hbm.at[pl.ds(off, CHUNK)], i_v, s)
        pltpu.async_copy(idxj_hbm.at[pl.ds(off, CHUNK)], j_v, s)

    def drain(slot):
        d_v, i_v, j_v, s = bufs[slot]
        pltpu.make_async_copy(dij_hbm.at[pl.ds(0, CHUNK)], d_v, s).wait()
        pltpu.make_async_copy(idxi_hbm.at[pl.ds(0, CHUNK)], i_v, s).wait()
        pltpu.make_async_copy(idxj_hbm.at[pl.ds(0, CHUNK)], j_v, s).wait()

    stripe = lax.iota(jnp.int32, 16) * VECS

    def compute(slot):
        d_v, i_v, j_v, _ = bufs[slot]

        @plsc.parallel_loop(0, VECS, unroll=4)
        def _vec(j):
            vidx = stripe + j
            d = plsc.load_gather(d_v, [vidx])
            ii = plsc.load_gather(i_v, [vidx])
            jj = plsc.load_gather(j_v, [vidx])
            qi = plsc.load_gather(qa_v, [ii])
            qj = plsc.load_gather(qa_v, [jj])
            x = d * INV_CUT
            x3 = x * x * x
            poly = x3 * ((6.0 * x - 15.0) * x + 10.0)
            sw = jnp.where(d < CUT, poly, 1.0)
            e_ord = 1.0 / d
            e_shield = _rsqrt16(d * d + 1.0)
            e = (0.5 * qi) * qj * ((1.0 - sw) * e_shield + sw * e_ord)
            plsc.addupdate_scatter(acc_v, [ii], e)

    pltpu.async_copy(qa_hbm, qa_v, qsem)
    fire(0, 0)
    zeros16 = jnp.zeros((16,), jnp.float32)

    @pl.loop(0, N_NODES // 16)
    def _zero(i):
        acc_v[pl.ds(i * 16, 16)] = zeros16

    pltpu.make_async_copy(qa_hbm, qa_v, qsem).wait()

    @pl.loop(0, (NCHUNKS - 1) // 2)
    def _pair(t):
        k = t * 2
        drain(0)
        fire(k + 1, 1)
        compute(0)
        drain(1)
        fire(k + 2, 0)
        compute(1)

    drain(0)
    compute(0)

    pltpu.sync_copy(acc_v, out_hbm.at[wid])


_sc_partials = pl.kernel(
    _sc_body,
    out_type=jax.ShapeDtypeStruct((NW, N_NODES), jnp.float32),
    mesh=plsc.VectorSubcoreMesh(
        core_axis_name="c", subcore_axis_name="s",
        num_cores=NUM_CORES, num_subcores=NUM_SUBCORES),
    scratch_types=[
        pltpu.VMEM((N_NODES,), jnp.float32),
        pltpu.VMEM((N_NODES,), jnp.float32),
        pltpu.VMEM((CHUNK,), jnp.float32),
        pltpu.VMEM((CHUNK,), jnp.float32),
        pltpu.VMEM((CHUNK,), jnp.int32),
        pltpu.VMEM((CHUNK,), jnp.int32),
        pltpu.VMEM((CHUNK,), jnp.int32),
        pltpu.VMEM((CHUNK,), jnp.int32),
        pltpu.SemaphoreType.DMA,
        pltpu.SemaphoreType.DMA,
        pltpu.SemaphoreType.DMA,
    ],
    compiler_params=pltpu.CompilerParams(needs_layout_passes=False),
)


def _merge_body(x_ref, o_ref):
    o_ref[...] = jnp.sum(x_ref[...], axis=0)


_merge = pl.pallas_call(
    _merge_body,
    out_shape=jax.ShapeDtypeStruct((N_NODES,), jnp.float32),
)


@jax.jit
def kernel(Z, Dij, Qa, idx_i, idx_j):
    partials = _sc_partials(Dij, Qa, idx_i, idx_j)
    eele = _merge(partials)
    return (eele, Qa)
